# Initial kernel scaffold; baseline (speedup 1.0000x reference)
import jax
import jax.numpy as jnp
from jax.experimental import pallas as pl


def kernel(x, edge_index, W1, b1, W2, b2, prelu_a):
    raise NotImplementedError("write your pallas kernel here")


# speedup vs baseline: 8.6972x; 8.6972x over previous
import functools

import jax
import jax.numpy as jnp
from jax import lax
from jax.experimental import pallas as pl
from jax.experimental.pallas import tpu as pltpu
from jax.experimental.pallas import tpu_sc as plsc

_NC = 2
_NS = 16
_L = 16
_K = 128


def _sc_mesh():
    return plsc.VectorSubcoreMesh(core_axis_name="c", subcore_axis_name="s",
                                  num_cores=_NC, num_subcores=_NS)


@functools.cache
def _deg_call(e_pad: int, r: int):
    ch = e_pad // (_NC * _NS * _K)
    rpt = r // _NS

    @functools.partial(
        pl.kernel,
        mesh=_sc_mesh(),
        out_type=jax.ShapeDtypeStruct((_NC, r, _L), jnp.float32),
        scratch_types=[
            pltpu.VMEM_SHARED((r, _L), jnp.float32),
            pltpu.VMEM((ch, _K), jnp.int32),
            pltpu.VMEM((_K, _L), jnp.float32),
            pltpu.VMEM((8, _L), jnp.float32),
        ],
    )
    def deg_kernel(dst_hbm, out_hbm, acc, idx_t, ones_v, zbuf):
        c = lax.axis_index("c")
        s = lax.axis_index("s")

        def fill_ones(i, _):
            ones_v[i, :] = jnp.full((_L,), 1.0, jnp.float32)
            return 0
        lax.fori_loop(0, _K, fill_ones, 0)
        for i in range(8):
            zbuf[i, :] = jnp.zeros((_L,), jnp.float32)

        def zero_acc(i, _):
            pltpu.sync_copy(zbuf, acc.at[pl.ds(s * rpt + i * 8, 8)])
            return 0
        lax.fori_loop(0, rpt // 8, zero_acc, 0)

        base = (c * _NS + s) * ch
        pltpu.sync_copy(dst_hbm.at[pl.ds(base, ch)], idx_t)
        plsc.subcore_barrier()

        def body(j, _):
            pltpu.sync_copy(ones_v, acc.at[idx_t.at[j]], add=True)
            return 0
        lax.fori_loop(0, ch, body, 0)
        plsc.subcore_barrier()

        pltpu.sync_copy(acc.at[pl.ds(s * rpt, rpt)],
                        out_hbm.at[c, pl.ds(s * rpt, rpt)])

    return deg_kernel


@functools.cache
def _agg_call(n: int, d: int, e_pad: int, r: int):
    ch = e_pad // (_NC * _NS * _K)
    rpt = r // _NS

    @functools.partial(
        pl.kernel,
        mesh=_sc_mesh(),
        out_type=jax.ShapeDtypeStruct((_NC, r, d), jnp.float32),
        scratch_types=[
            pltpu.VMEM_SHARED((r, d), jnp.float32),
            pltpu.VMEM((ch, _K), jnp.int32),
            pltpu.VMEM((ch, _K), jnp.int32),
            pltpu.VMEM((_K, d), jnp.float32),
            pltpu.VMEM((8, d), jnp.float32),
            pltpu.SemaphoreType.DMA,
        ],
    )
    def agg_kernel(h_hbm, src_hbm, dst_hbm, out_hbm,
                   acc, src_t, dst_t, rows, zbuf, sem):
        c = lax.axis_index("c")
        s = lax.axis_index("s")

        for i in range(8):
            for j in range(d // _L):
                zbuf[i, pl.ds(j * _L, _L)] = jnp.zeros((_L,), jnp.float32)

        def zero_acc(i, _):
            pltpu.sync_copy(zbuf, acc.at[pl.ds(s * rpt + i * 8, 8)])
            return 0
        lax.fori_loop(0, rpt // 8, zero_acc, 0)

        base = (c * _NS + s) * ch
        pltpu.sync_copy(src_hbm.at[pl.ds(base, ch)], src_t)
        pltpu.sync_copy(dst_hbm.at[pl.ds(base, ch)], dst_t)
        plsc.subcore_barrier()

        def body(j, _):
            pltpu.async_copy(h_hbm.at[src_t.at[j]], rows, sem).wait()
            pltpu.sync_copy(rows, acc.at[dst_t.at[j]], add=True)
            return 0
        lax.fori_loop(0, ch, body, 0)
        plsc.subcore_barrier()

        pltpu.sync_copy(acc.at[pl.ds(s * rpt, rpt)],
                        out_hbm.at[c, pl.ds(s * rpt, rpt)])

    return agg_kernel


def _dinv_of(degp_blk):
    deg = 1.0 + degp_blk[0, :, :1] + degp_blk[1, :, :1]
    return lax.rsqrt(deg)


def _mm1_body(x_ref, w_ref, degp_ref, o_ref):
    dinv = _dinv_of(degp_ref[...])
    h = jnp.dot(x_ref[...], w_ref[...],
                preferred_element_type=jnp.float32,
                precision=lax.Precision.HIGHEST)
    o_ref[...] = h * dinv


def _mid_body(p_ref, h1_ref, degp_ref, w_ref, b_ref, a_ref, o_ref):
    dinv = _dinv_of(degp_ref[...])
    g = (p_ref[0] + p_ref[1] + h1_ref[...]) * dinv + b_ref[...]
    t = jnp.maximum(g, 0.0) + a_ref[...] * jnp.minimum(g, 0.0)
    h2 = jnp.dot(t, w_ref[...],
                 preferred_element_type=jnp.float32,
                 precision=lax.Precision.HIGHEST)
    o_ref[...] = h2 * dinv


def _fin_body(p_ref, h2_ref, degp_ref, b_ref, o_ref):
    dinv = _dinv_of(degp_ref[...])
    o_ref[...] = (p_ref[0] + p_ref[1] + h2_ref[...]) * dinv + b_ref[...]


def _row_specs(m_blk, d, r):
    node = pl.BlockSpec((m_blk, d), lambda i: (i, 0))
    part = pl.BlockSpec((2, m_blk, d), lambda i: (0, i, 0))
    degp = pl.BlockSpec((2, m_blk, _L), lambda i: (0, i, 0))
    full = pl.BlockSpec((d, d), lambda i: (0, 0))
    vec = pl.BlockSpec((1, d), lambda i: (0, 0))
    return node, part, degp, full, vec


def _tc_calls(n: int, d: int, r: int, m_blk: int):
    node, part, degp, full, vec = _row_specs(m_blk, d, r)
    grid = (n // m_blk,)
    out = jax.ShapeDtypeStruct((n, d), jnp.float32)
    mm1 = pl.pallas_call(
        _mm1_body, grid=grid, out_shape=out,
        in_specs=[node, full, degp], out_specs=node)
    mid = pl.pallas_call(
        _mid_body, grid=grid, out_shape=out,
        in_specs=[part, node, degp, full, vec, vec], out_specs=node)
    fin = pl.pallas_call(
        _fin_body, grid=grid, out_shape=out,
        in_specs=[part, node, degp, vec], out_specs=node)
    return mm1, mid, fin


def kernel(x, edge_index, W1, b1, W2, b2, prelu_a):
    n, d = x.shape
    e = edge_index.shape[1]
    ept = -(-e // (_NC * _NS * _K * 8)) * _K * 8
    e_pad = ept * _NC * _NS
    r = -(-(n + 1) // (8 * _NS)) * 8 * _NS

    src = edge_index[0].astype(jnp.int32)
    dst = edge_index[1].astype(jnp.int32)
    pad = e_pad - e
    src_p = jnp.concatenate([src, jnp.zeros((pad,), jnp.int32)])
    src_p = src_p.reshape(e_pad // _K, _K)
    dst_p = jnp.concatenate([dst, jnp.full((pad,), n, jnp.int32)])
    dst_p = dst_p.reshape(e_pad // _K, _K)

    mm1, mid, fin = _tc_calls(n, d, r, m_blk=2000)

    degp = _deg_call(e_pad, r)(dst_p)
    h1 = mm1(x, W1, degp)
    p1 = _agg_call(n, d, e_pad, r)(h1, src_p, dst_p)
    h2 = mid(p1, h1, degp, W2, b1.reshape(1, d), prelu_a.reshape(1, d))
    p2 = _agg_call(n, d, e_pad, r)(h2, src_p, dst_p)
    return fin(p2, h2, degp, b2.reshape(1, d))


# speedup vs baseline: 9.5983x; 1.1036x over previous
import functools

import jax
import jax.numpy as jnp
from jax import lax
from jax.experimental import pallas as pl
from jax.experimental.pallas import tpu as pltpu
from jax.experimental.pallas import tpu_sc as plsc

_NC = 2
_NS = 16
_L = 16
_K = 128


def _sc_mesh():
    return plsc.VectorSubcoreMesh(core_axis_name="c", subcore_axis_name="s",
                                  num_cores=_NC, num_subcores=_NS)


@functools.cache
def _deg_call(e_pad: int, r: int):
    ch = e_pad // (_NC * _NS * _K)
    rpt = r // _NS

    @functools.partial(
        pl.kernel,
        mesh=_sc_mesh(),
        out_type=jax.ShapeDtypeStruct((_NC, r, _L), jnp.float32),
        scratch_types=[
            pltpu.VMEM_SHARED((r, _L), jnp.float32),
            pltpu.VMEM((ch, _K), jnp.int32),
            pltpu.VMEM((_K, _L), jnp.float32),
            pltpu.VMEM((8, _L), jnp.float32),
        ],
    )
    def deg_kernel(dst_hbm, out_hbm, acc, idx_t, ones_v, zbuf):
        c = lax.axis_index("c")
        s = lax.axis_index("s")

        def fill_ones(i, _):
            ones_v[i, :] = jnp.full((_L,), 1.0, jnp.float32)
            return 0
        lax.fori_loop(0, _K, fill_ones, 0)
        for i in range(8):
            zbuf[i, :] = jnp.zeros((_L,), jnp.float32)

        def zero_acc(i, _):
            pltpu.sync_copy(zbuf, acc.at[pl.ds(s * rpt + i * 8, 8)])
            return 0
        lax.fori_loop(0, rpt // 8, zero_acc, 0)

        base = (c * _NS + s) * ch
        pltpu.sync_copy(dst_hbm.at[pl.ds(base, ch)], idx_t)
        plsc.subcore_barrier()

        def body(j, _):
            pltpu.sync_copy(ones_v, acc.at[idx_t.at[j]], add=True)
            return 0
        lax.fori_loop(0, ch, body, 0)
        plsc.subcore_barrier()

        pltpu.sync_copy(acc.at[pl.ds(s * rpt, rpt)],
                        out_hbm.at[c, pl.ds(s * rpt, rpt)])

    return deg_kernel


@functools.cache
def _agg_call(n: int, d: int, e_pad: int, r: int):
    ch = e_pad // (_NC * _NS * _K)
    rpt = r // _NS

    nblk = ch // 8

    @functools.partial(
        pl.kernel,
        mesh=_sc_mesh(),
        out_type=jax.ShapeDtypeStruct((_NC, r, d), jnp.float32),
        scratch_types=[
            pltpu.VMEM_SHARED((r, d), jnp.float32),
            pltpu.VMEM((ch, _K), jnp.int32),
            pltpu.VMEM((2, 8, _K), jnp.int32),
            pltpu.VMEM((2, _K, d), jnp.float32),
            pltpu.SemaphoreType.DMA((2,)),
            pltpu.SemaphoreType.DMA((2,)),
        ],
    )
    def agg_kernel(h_hbm, src_hbm, dst_hbm, out_hbm,
                   acc, src_t, dring, rows, gsem, dsem):
        c = lax.axis_index("c")
        s = lax.axis_index("s")

        def zrow(i, _):
            for jj in range(d // _L):
                rows[0, i, pl.ds(jj * _L, _L)] = jnp.zeros((_L,), jnp.float32)
            return 0
        lax.fori_loop(0, _K, zrow, 0)

        def zero_acc(i, _):
            pltpu.sync_copy(rows.at[0], acc.at[pl.ds(s * rpt + i * _K, _K)])
            return 0
        lax.fori_loop(0, rpt // _K, zero_acc, 0)

        base = (c * _NS + s) * ch
        pltpu.sync_copy(src_hbm.at[pl.ds(base, ch)], src_t)
        plsc.subcore_barrier()

        for blk in range(2):
            pltpu.async_copy(dst_hbm.at[pl.ds(base + blk * 8, 8)],
                             dring.at[blk], dsem.at[blk])
        for b in range(2):
            pltpu.async_copy(h_hbm.at[src_t.at[b]], rows.at[b], gsem.at[b])

        def super_group(gi, _):
            for blk in range(2):
                bidx = gi * 2 + blk
                pltpu.make_async_copy(dst_hbm.at[pl.ds(0, 8)],
                                      dring.at[blk], dsem.at[blk]).wait()
                for bb in range(8):
                    j = bidx * 8 + bb
                    b = bb % 2
                    pltpu.make_async_copy(
                        h_hbm.at[src_t.at[j]], rows.at[b], gsem.at[b]).wait()
                    pltpu.sync_copy(rows.at[b], acc.at[dring.at[blk, bb]],
                                    add=True)
                    nxt = j + 2

                    @pl.when(nxt < ch)
                    def _():
                        pltpu.async_copy(
                            h_hbm.at[src_t.at[nxt]], rows.at[b], gsem.at[b])
                nxtb = bidx + 2

                @pl.when(nxtb < nblk)
                def _():
                    pltpu.async_copy(dst_hbm.at[pl.ds(base + nxtb * 8, 8)],
                                     dring.at[blk], dsem.at[blk])
            return 0
        lax.fori_loop(0, nblk // 2, super_group, 0)
        plsc.subcore_barrier()

        pltpu.sync_copy(acc.at[pl.ds(s * rpt, rpt)],
                        out_hbm.at[c, pl.ds(s * rpt, rpt)])

    return agg_kernel


def _dinv_of(degp_blk):
    deg = 1.0 + degp_blk[0, :, :1] + degp_blk[1, :, :1]
    return lax.rsqrt(deg)


def _mm1_body(x_ref, w_ref, degp_ref, o_ref):
    dinv = _dinv_of(degp_ref[...])
    h = jnp.dot(x_ref[...], w_ref[...],
                preferred_element_type=jnp.float32,
                precision=lax.Precision.HIGHEST)
    o_ref[...] = h * dinv


def _mid_body(p_ref, h1_ref, degp_ref, w_ref, b_ref, a_ref, o_ref):
    dinv = _dinv_of(degp_ref[...])
    g = (p_ref[0] + p_ref[1] + h1_ref[...]) * dinv + b_ref[...]
    t = jnp.maximum(g, 0.0) + a_ref[...] * jnp.minimum(g, 0.0)
    h2 = jnp.dot(t, w_ref[...],
                 preferred_element_type=jnp.float32,
                 precision=lax.Precision.HIGHEST)
    o_ref[...] = h2 * dinv


def _fin_body(p_ref, h2_ref, degp_ref, b_ref, o_ref):
    dinv = _dinv_of(degp_ref[...])
    o_ref[...] = (p_ref[0] + p_ref[1] + h2_ref[...]) * dinv + b_ref[...]


def _row_specs(m_blk, d, r):
    node = pl.BlockSpec((m_blk, d), lambda i: (i, 0))
    part = pl.BlockSpec((2, m_blk, d), lambda i: (0, i, 0))
    degp = pl.BlockSpec((2, m_blk, _L), lambda i: (0, i, 0))
    full = pl.BlockSpec((d, d), lambda i: (0, 0))
    vec = pl.BlockSpec((1, d), lambda i: (0, 0))
    return node, part, degp, full, vec


def _tc_calls(n: int, d: int, r: int, m_blk: int):
    node, part, degp, full, vec = _row_specs(m_blk, d, r)
    grid = (n // m_blk,)
    out = jax.ShapeDtypeStruct((n, d), jnp.float32)
    mm1 = pl.pallas_call(
        _mm1_body, grid=grid, out_shape=out,
        in_specs=[node, full, degp], out_specs=node)
    mid = pl.pallas_call(
        _mid_body, grid=grid, out_shape=out,
        in_specs=[part, node, degp, full, vec, vec], out_specs=node)
    fin = pl.pallas_call(
        _fin_body, grid=grid, out_shape=out,
        in_specs=[part, node, degp, vec], out_specs=node)
    return mm1, mid, fin


def kernel(x, edge_index, W1, b1, W2, b2, prelu_a):
    n, d = x.shape
    e = edge_index.shape[1]
    ept = -(-e // (_NC * _NS * _K * 8)) * _K * 8
    e_pad = ept * _NC * _NS
    r = -(-(n + 1) // (_K * _NS)) * _K * _NS

    src = edge_index[0].astype(jnp.int32)
    dst = edge_index[1].astype(jnp.int32)
    pad = e_pad - e
    src_p = jnp.concatenate([src, jnp.zeros((pad,), jnp.int32)])
    src_p = src_p.reshape(e_pad // _K, _K)
    dst_p = jnp.concatenate([dst, jnp.full((pad,), n, jnp.int32)])
    dst_p = dst_p.reshape(e_pad // _K, _K)

    mm1, mid, fin = _tc_calls(n, d, r, m_blk=2000)

    degp = _deg_call(e_pad, r)(dst_p)
    h1 = mm1(x, W1, degp)
    p1 = _agg_call(n, d, e_pad, r)(h1, src_p, dst_p)
    h2 = mid(p1, h1, degp, W2, b1.reshape(1, d), prelu_a.reshape(1, d))
    p2 = _agg_call(n, d, e_pad, r)(h2, src_p, dst_p)
    return fin(p2, h2, degp, b2.reshape(1, d))
